# baseline (speedup 1.0000x reference)
import functools

import jax
import jax.numpy as jnp
from jax import lax
from jax.experimental import pallas as pl
from jax.experimental.pallas import tpu as pltpu
from jax.experimental.pallas import tpu_sc as plsc

L = 16


def _build(B, S, H, NC, NS):
    NW = NC * NS
    assert S % NW == 0 and H % L == 0
    BAND = S // NW
    NJ = H // L
    NB = B
    assert NB % 2 == 0 and BAND % 8 == 0

    mesh = plsc.VectorSubcoreMesh(core_axis_name="c", subcore_axis_name="s")

    @functools.partial(
        pl.kernel,
        mesh=mesh,
        out_type=jax.ShapeDtypeStruct((B * S, H), jnp.float32),
        scratch_types=[
            pltpu.VMEM((BAND, H), jnp.float32),
            pltpu.VMEM((H,), jnp.float32),
            pltpu.VMEM((H,), jnp.float32),
            pltpu.VMEM((BAND,), jnp.int32),
            pltpu.VMEM((BAND,), jnp.int32),
            pltpu.VMEM((BAND, H), jnp.float32),
            pltpu.VMEM((BAND, H), jnp.float32),
            pltpu.VMEM((BAND, H), jnp.float32),
            pltpu.VMEM((BAND, H), jnp.float32),
            pltpu.VMEM((2 * L,), jnp.float32),
            pltpu.SemaphoreType.DMA,
            pltpu.SemaphoreType.DMA,
            pltpu.SemaphoreType.DMA,
            pltpu.SemaphoreType.DMA,
        ],
    )
    def emb_ln(ids_hbm, table_hbm, pos_hbm, gamma_hbm, beta_hbm, out_hbm,
               pos_v, gamma_v, beta_v, idx0, idx1, tok0, tok1, out0, out1,
               stats_v, gsem0, gsem1, osem0, osem1):
        c = lax.axis_index("c")
        s = lax.axis_index("s")
        wid = s * NC + c
        s0 = wid * BAND

        pltpu.sync_copy(pos_hbm.at[pl.ds(s0, BAND)], pos_v)
        pltpu.sync_copy(gamma_hbm, gamma_v)
        pltpu.sync_copy(beta_hbm, beta_v)

        idx = (idx0, idx1)
        tok = (tok0, tok1)
        outb = (out0, out1)
        gsem = (gsem0, gsem1)
        osem = (osem0, osem1)

        def start_gather(g, p):
            base = g * S + s0
            pltpu.sync_copy(ids_hbm.at[pl.ds(base, BAND)], idx[p])
            pltpu.async_copy(table_hbm.at[idx[p]], tok[p], gsem[p])

        def compute_row(r, tok_ref, out_ref):
            zero = jnp.zeros((L,), jnp.float32)
            acc = [zero, zero, zero, zero]
            asq = [zero, zero, zero, zero]
            for j in range(NJ):
                sl = pl.ds(j * L, L)
                x = tok_ref[r, sl] + pos_v[r, sl]
                out_ref[r, sl] = x
                k = j % 4
                acc[k] = acc[k] + x
                asq[k] = asq[k] + x * x
            def lanesum(vec):
                parts = [vec[i] for i in range(L)]
                while len(parts) > 1:
                    parts = [parts[i] + parts[i + 1]
                             for i in range(0, len(parts), 2)]
                return parts[0]
            tot = lanesum((acc[0] + acc[1]) + (acc[2] + acc[3]))
            tsq = lanesum((asq[0] + asq[1]) + (asq[2] + asq[3]))
            mean = tot * (1.0 / H)
            var = tsq * (1.0 / H) - mean * mean
            var = jnp.maximum(var, 0.0) + 1e-12
            iv = lax.bitcast_convert_type(var, jnp.int32)
            yi = jnp.int32(0x5F3759DF) - lax.shift_right_logical(iv, 1)
            y = lax.bitcast_convert_type(yi, jnp.float32)
            vh = var * 0.5
            for _ in range(3):
                y = y * (1.5 - vh * y * y)
            inv_v = jnp.full((L,), y, jnp.float32)
            mean_v = jnp.full((L,), mean, jnp.float32)
            for j in range(NJ):
                sl = pl.ds(j * L, L)
                xx = out_ref[r, sl]
                out_ref[r, sl] = (xx - mean_v) * inv_v * gamma_v[sl] + beta_v[sl]

        def chunk(g, p):
            pltpu.make_async_copy(table_hbm.at[idx[p]], tok[p], gsem[p]).wait()

            @pl.when(g >= 2)
            def _():
                pltpu.make_async_copy(
                    outb[p], out_hbm.at[pl.ds(0, BAND)], osem[p]).wait()

            def row(r, carry):
                compute_row(r, tok[p], outb[p])
                return carry
            lax.fori_loop(0, BAND, row, 0)

            base = g * S + s0
            pltpu.async_copy(outb[p], out_hbm.at[pl.ds(base, BAND)], osem[p])

            @pl.when(g + 2 < NB)
            def _():
                start_gather(g + 2, p)

        start_gather(0, 0)
        start_gather(1, 1)

        def outer(gg, carry):
            chunk(gg * 2, 0)
            chunk(gg * 2 + 1, 1)
            return carry
        lax.fori_loop(0, NB // 2, outer, 0)

        pltpu.make_async_copy(outb[0], out_hbm.at[pl.ds(0, BAND)], osem[0]).wait()
        pltpu.make_async_copy(outb[1], out_hbm.at[pl.ds(0, BAND)], osem[1]).wait()

    return emb_ln


def kernel(ipt_ids, token_table, pos_table, gamma, beta):
    B, S = ipt_ids.shape
    H = token_table.shape[1]
    info = plsc.get_sparse_core_info()
    NC, NS = info.num_cores, info.num_subcores
    ids = ipt_ids.reshape(B * S).astype(jnp.int32)
    fn = _build(B, S, H, NC, NS)
    out = fn(ids, token_table, pos_table, gamma, beta)
    return out.reshape(B, S, H)
